# baseline (speedup 1.0000x reference)
import functools

import jax
import jax.numpy as jnp
from jax import lax
from jax.experimental import pallas as pl
from jax.experimental.pallas import tpu as pltpu
from jax.experimental.pallas import tpu_sc as plsc

BATCH = 4
SEQ = 8192
N = BATCH * SEQ
D = 1024
NW = 32
ROWS_PER_W = N // NW
CHUNK = 32
NCHUNKS = ROWS_PER_W // CHUNK


def _make_kernel():
    mesh = plsc.VectorSubcoreMesh(core_axis_name="c", subcore_axis_name="s")

    @functools.partial(
        pl.kernel,
        mesh=mesh,
        out_type=jax.ShapeDtypeStruct((N, D), jnp.float32),
        scratch_types=[
            pltpu.VMEM((ROWS_PER_W,), jnp.int32),
            pltpu.VMEM((CHUNK, D), jnp.float32),
            pltpu.VMEM((CHUNK, D), jnp.float32),
            pltpu.SemaphoreType.DMA,
            pltpu.SemaphoreType.DMA,
            pltpu.SemaphoreType.DMA,
            pltpu.SemaphoreType.DMA,
        ],
    )
    def k(ids_hbm, table_hbm, out_hbm, idx_v, buf0, buf1,
          gsem0, gsem1, ssem0, ssem1):
        wid = lax.axis_index("s") * 2 + lax.axis_index("c")
        base = wid * ROWS_PER_W
        pltpu.sync_copy(ids_hbm.at[pl.ds(base, ROWS_PER_W)], idx_v)

        bufs = (buf0, buf1)
        gsems = (gsem0, gsem1)
        ssems = (ssem0, ssem1)

        def gather(g):
            b = g % 2
            return pltpu.async_copy(
                table_hbm.at[idx_v.at[pl.ds(g * CHUNK, CHUNK)]],
                bufs[b], gsems[b])

        def scatter(g):
            b = g % 2
            return pltpu.async_copy(
                bufs[b], out_hbm.at[pl.ds(base + g * CHUNK, CHUNK)],
                ssems[b])

        scatters = [None] * NCHUNKS
        gathers = [None] * NCHUNKS
        gathers[0] = gather(0)
        for g in range(NCHUNKS):
            if g + 1 < NCHUNKS:
                if g >= 1:
                    scatters[g - 1].wait()
                gathers[g + 1] = gather(g + 1)
            gathers[g].wait()
            scatters[g] = scatter(g)
        scatters[NCHUNKS - 2].wait()
        scatters[NCHUNKS - 1].wait()

    return k


_k = _make_kernel()


def kernel(token_type_ids, table):
    ids_flat = token_type_ids.reshape(-1).astype(jnp.int32)
    out = _k(ids_flat, table.astype(jnp.float32))
    return out.reshape(BATCH, SEQ, D)


# speedup vs baseline: 13.4902x; 13.4902x over previous
import functools

import jax
import jax.numpy as jnp
from jax import lax
from jax.experimental import pallas as pl
from jax.experimental.pallas import tpu as pltpu
from jax.experimental.pallas import tpu_sc as plsc

BATCH = 4
SEQ = 8192
N = BATCH * SEQ
D = 1024
NW = 32
ROWS_PER_W = N // NW
UNROLL = 16
NBLK = ROWS_PER_W // UNROLL


def _make_kernel():
    mesh = plsc.VectorSubcoreMesh(core_axis_name="c", subcore_axis_name="s")

    @functools.partial(
        pl.kernel,
        mesh=mesh,
        out_type=jax.ShapeDtypeStruct((N, D), jnp.float32),
        scratch_types=[
            pltpu.VMEM((ROWS_PER_W,), jnp.int32),
            pltpu.VMEM((2, D), jnp.float32),
            pltpu.SemaphoreType.DMA,
        ],
    )
    def k(ids_hbm, table_hbm, out_hbm, idx_v, tab_v, sem):
        wid = lax.axis_index("s") * 2 + lax.axis_index("c")
        base = wid * ROWS_PER_W
        pltpu.sync_copy(ids_hbm.at[pl.ds(base, ROWS_PER_W)], idx_v)
        pltpu.sync_copy(table_hbm, tab_v)

        def body(blk, _):
            r0 = blk * UNROLL
            v = idx_v[pl.ds(r0, 16)]
            for j in range(UNROLL):
                t = v[j]
                pltpu.async_copy(tab_v.at[t], out_hbm.at[base + r0 + j], sem)
            return _

        lax.fori_loop(0, NBLK, body, None)
        pltpu.make_async_copy(
            out_hbm.at[pl.ds(base, ROWS_PER_W)],
            out_hbm.at[pl.ds(base, ROWS_PER_W)],
            sem,
        ).wait()

    return k


_k = _make_kernel()


def kernel(token_type_ids, table):
    ids_flat = token_type_ids.reshape(-1).astype(jnp.int32)
    out = _k(ids_flat, table.astype(jnp.float32))
    return out.reshape(BATCH, SEQ, D)
